# baseline (speedup 1.0000x reference)
import jax
import jax.numpy as jnp
from jax.experimental import pallas as pl
from jax.experimental.pallas import tpu as pltpu
from jax.experimental.pallas import tpu_sc as plsc

N_TASKS = 8
D_IN = 2048
D_OUT = 2048
D_H = 1024
D_LANG = 768
B = 4096

TB = 256
TB_LOG2 = 8
NB_MAX = B // TB + N_TASKS
P = NB_MAX * TB
META_LEN = 32

_SC_MESH = plsc.VectorSubcoreMesh(core_axis_name="c", subcore_axis_name="s")
_NW = 32


def _router_body(lt_ref, emb_ref, out_ref):
    lt = lt_ref[...]
    emb = emb_ref[...]
    nt = lt / jnp.maximum(
        jnp.sqrt(jnp.sum(lt * lt, axis=1, keepdims=True)), 1e-12)
    ne = emb / jnp.maximum(
        jnp.sqrt(jnp.sum(emb * emb, axis=1, keepdims=True)), 1e-12)
    sims = jax.lax.dot_general(
        nt, ne, (((1,), (1,)), ((), ())),
        preferred_element_type=jnp.float32)
    idx = jnp.argmax(sims, axis=1).astype(jnp.int32)
    out_ref[...] = idx[None, None, :]


def _route(language_token, route_embeddings):
    rb = 1024
    nblk = B // rb
    out = pl.pallas_call(
        _router_body,
        grid=(nblk,),
        in_specs=[
            pl.BlockSpec((rb, D_LANG), lambda i: (i, 0)),
            pl.BlockSpec((N_TASKS, D_LANG), lambda i: (0, 0)),
        ],
        out_specs=pl.BlockSpec((1, 1, rb), lambda i: (i, 0, 0)),
        out_shape=jax.ShapeDtypeStruct((nblk, 1, rb), jnp.int32),
    )(language_token, route_embeddings)
    return out.reshape(B)


def _pfx16(x):
    lanes = jax.lax.iota(jnp.int32, 16)
    for k in (1, 2, 4, 8):
        sh = x.at[jnp.maximum(lanes - k, 0)].get(mode="promise_in_bounds")
        x = x + jnp.where(lanes >= k, sh, 0)
    return x


def _bcast16(x, k):
    return x.at[jnp.full((16,), k, jnp.int32)].get(mode="promise_in_bounds")


def _sort_body(idx_hbm, inv_hbm, meta_hbm, idx_v, inv_v, meta_v):
    cid = jax.lax.axis_index("c")
    sid = jax.lax.axis_index("s")

    @pl.when(jnp.logical_and(cid == 0, sid == 0))
    def _():
        pltpu.sync_copy(idx_hbm, idx_v)
        lanes = jax.lax.iota(jnp.int32, 16)
        zeros = jnp.zeros((16,), jnp.int32)
        ones = jnp.ones((16,), jnp.int32)

        def h_body(c, accs):
            v = idx_v[pl.ds(c * 16, 16)]
            return tuple(acc + jnp.where(v == b, 1, 0)
                         for b, acc in enumerate(accs))

        accs = jax.lax.fori_loop(0, B // 16, h_body, (zeros,) * N_TASKS)
        counts = zeros
        for b in range(N_TASKS):
            tot = _bcast16(_pfx16(accs[b]), 15)
            counts = jnp.where(lanes == b, tot, counts)

        nblk = (counts + (TB - 1)) >> TB_LOG2
        bsi = _pfx16(nblk)
        pad_off = (bsi - nblk) * TB
        num_used_v = _bcast16(bsi, N_TASKS - 1)

        bsi_e = [_bcast16(bsi, e) for e in range(N_TASKS)]
        acc0 = zeros
        acc1 = zeros
        kvec1 = lanes + 16
        for e in range(N_TASKS):
            acc0 = acc0 + jnp.where(lanes >= bsi_e[e], 1, 0)
            acc1 = acc1 + jnp.where(kvec1 >= bsi_e[e], 1, 0)
        meta_v[pl.ds(0, 16)] = jnp.minimum(acc0, N_TASKS - 1)
        meta_v[pl.ds(16, 16)] = jnp.where(
            lanes == (NB_MAX - 16), num_used_v, jnp.minimum(acc1, N_TASKS - 1))

        def s_body(c, offs):
            v = idx_v[pl.ds(c * 16, 16)]
            lo = v < 4
            s0 = jnp.where(lo, v, 0) * 8
            s1 = jnp.where(lo, 0, v - 4) * 8
            w0 = jnp.where(lo, ones << s0, 0)
            w1 = jnp.where(lo, 0, ones << s1)
            p0 = _pfx16(w0)
            p1 = _pfx16(w1)
            psel = jnp.where(lo, p0, p1)
            rank = ((psel >> jnp.where(lo, s0, s1)) & 255) - 1
            off_v = offs.at[v].get(mode="promise_in_bounds")
            inv_v[pl.ds(c * 16, 16)] = off_v + rank
            t0 = _bcast16(p0, 15)
            t1 = _bcast16(p1, 15)
            c0 = (t0 >> jnp.minimum(lanes, 3) * 8) & 255
            c1 = (t1 >> (jnp.minimum(jnp.maximum(lanes - 4, 0), 3) * 8)) & 255
            cnt = jnp.where(lanes < 4, c0, jnp.where(lanes < 8, c1, 0))
            return offs + cnt

        jax.lax.fori_loop(0, B // 16, s_body, pad_off)

        pltpu.sync_copy(inv_v, inv_hbm)
        pltpu.sync_copy(meta_v, meta_hbm)


_sort = pl.kernel(
    _sort_body,
    out_type=[
        jax.ShapeDtypeStruct((B,), jnp.int32),
        jax.ShapeDtypeStruct((META_LEN,), jnp.int32),
    ],
    mesh=_SC_MESH,
    scratch_types=[
        pltpu.VMEM((B,), jnp.int32),
        pltpu.VMEM((B,), jnp.int32),
        pltpu.VMEM((META_LEN,), jnp.int32),
    ],
)


def _scatter_x_body(x_hbm, inv_hbm, xs_hbm, idx_c, rows_v, sem):
    cid = jax.lax.axis_index("c")
    sid = jax.lax.axis_index("s")
    base = (sid * 2 + cid) * (B // _NW)

    def c_body(j, carry):
        b2 = base + j * 32
        pltpu.sync_copy(inv_hbm.at[pl.ds(b2, 32)], idx_c)
        pltpu.sync_copy(x_hbm.at[pl.ds(b2, 32)], rows_v)
        pltpu.async_copy(rows_v, xs_hbm.at[idx_c], sem).wait()
        return carry

    jax.lax.fori_loop(0, (B // _NW) // 32, c_body, 0)


_scatter_x = pl.kernel(
    _scatter_x_body,
    out_type=jax.ShapeDtypeStruct((P, D_IN), jnp.float32),
    mesh=_SC_MESH,
    scratch_types=[
        pltpu.VMEM((32,), jnp.int32),
        pltpu.VMEM((32, D_IN), jnp.float32),
        pltpu.SemaphoreType.DMA,
    ],
)


def _gather_y_body(ys_hbm, inv_hbm, y_hbm, idx_c, rows_v, sem):
    cid = jax.lax.axis_index("c")
    sid = jax.lax.axis_index("s")
    base = (sid * 2 + cid) * (B // _NW)

    def c_body(j, carry):
        b2 = base + j * 32
        pltpu.sync_copy(inv_hbm.at[pl.ds(b2, 32)], idx_c)
        pltpu.async_copy(ys_hbm.at[idx_c], rows_v, sem).wait()
        pltpu.sync_copy(rows_v, y_hbm.at[pl.ds(b2, 32)])
        return carry

    jax.lax.fori_loop(0, (B // _NW) // 32, c_body, 0)


_gather_y = pl.kernel(
    _gather_y_body,
    out_type=jax.ShapeDtypeStruct((B, D_OUT), jnp.float32),
    mesh=_SC_MESH,
    scratch_types=[
        pltpu.VMEM((32,), jnp.int32),
        pltpu.VMEM((32, D_OUT), jnp.float32),
        pltpu.SemaphoreType.DMA,
    ],
)


def _mlp_body(meta_ref, x_ref, w1_ref, b1_ref, w2_ref, b2_ref, w3_ref, b3_ref,
              out_ref):
    i = pl.program_id(0)

    @pl.when(i < meta_ref[NB_MAX])
    def _():
        xb = x_ref[...].astype(jnp.bfloat16)
        h = jnp.maximum(
            jnp.dot(xb, w1_ref[0], preferred_element_type=jnp.float32)
            + b1_ref[0], 0.0)
        h = jnp.maximum(
            jnp.dot(h.astype(jnp.bfloat16), w2_ref[0],
                    preferred_element_type=jnp.float32)
            + b2_ref[0], 0.0)
        out_ref[...] = jnp.tanh(
            jnp.dot(h.astype(jnp.bfloat16), w3_ref[0],
                    preferred_element_type=jnp.float32)
            + b3_ref[0])


def _grouped_mlp(meta, x_sorted, W1, b1, W2, b2, W3, b3):
    grid_spec = pltpu.PrefetchScalarGridSpec(
        num_scalar_prefetch=1,
        grid=(NB_MAX,),
        in_specs=[
            pl.BlockSpec((TB, D_IN), lambda i, m: (i, 0)),
            pl.BlockSpec((1, D_IN, D_H), lambda i, m: (m[i], 0, 0)),
            pl.BlockSpec((1, 1, D_H), lambda i, m: (m[i], 0, 0)),
            pl.BlockSpec((1, D_H, D_H), lambda i, m: (m[i], 0, 0)),
            pl.BlockSpec((1, 1, D_H), lambda i, m: (m[i], 0, 0)),
            pl.BlockSpec((1, D_H, D_OUT), lambda i, m: (m[i], 0, 0)),
            pl.BlockSpec((1, 1, D_OUT), lambda i, m: (m[i], 0, 0)),
        ],
        out_specs=pl.BlockSpec((TB, D_OUT), lambda i, m: (i, 0)),
    )
    return pl.pallas_call(
        _mlp_body,
        grid_spec=grid_spec,
        out_shape=jax.ShapeDtypeStruct((P, D_OUT), jnp.float32),
        compiler_params=pltpu.CompilerParams(
            vmem_limit_bytes=100 * 1024 * 1024),
    )(meta, x_sorted,
      W1.astype(jnp.bfloat16), b1.reshape(N_TASKS, 1, D_H),
      W2.astype(jnp.bfloat16), b2.reshape(N_TASKS, 1, D_H),
      W3.astype(jnp.bfloat16), b3.reshape(N_TASKS, 1, D_OUT))


def kernel(x, language_token, route_embeddings, W1, b1, W2, b2, W3, b3):
    indices = _route(language_token, route_embeddings)
    inv, meta = _sort(indices)
    x_sorted = _scatter_x(x, inv)
    y_sorted = _grouped_mlp(meta, x_sorted, W1, b1, W2, b2, W3, b3)
    return _gather_y(y_sorted, inv)


# speedup vs baseline: 1.2198x; 1.2198x over previous
import jax
import jax.numpy as jnp
from jax.experimental import pallas as pl
from jax.experimental.pallas import tpu as pltpu
from jax.experimental.pallas import tpu_sc as plsc

N_TASKS = 8
D_IN = 2048
D_OUT = 2048
D_H = 1024
D_LANG = 768
B = 4096

TB = 256
TB_LOG2 = 8
NB_MAX = B // TB + N_TASKS
P = NB_MAX * TB
META_LEN = 32

_SC_MESH = plsc.VectorSubcoreMesh(core_axis_name="c", subcore_axis_name="s")
_NW = 32


def _router_body(lt_ref, emb_ref, out_ref):
    lt = lt_ref[...]
    emb = emb_ref[...]
    nt = lt / jnp.maximum(
        jnp.sqrt(jnp.sum(lt * lt, axis=1, keepdims=True)), 1e-12)
    ne = emb / jnp.maximum(
        jnp.sqrt(jnp.sum(emb * emb, axis=1, keepdims=True)), 1e-12)
    sims = jax.lax.dot_general(
        nt, ne, (((1,), (1,)), ((), ())),
        preferred_element_type=jnp.float32)
    idx = jnp.argmax(sims, axis=1).astype(jnp.int32)
    out_ref[...] = idx[None, None, :]


def _route(language_token, route_embeddings):
    rb = 1024
    nblk = B // rb
    out = pl.pallas_call(
        _router_body,
        grid=(nblk,),
        in_specs=[
            pl.BlockSpec((rb, D_LANG), lambda i: (i, 0)),
            pl.BlockSpec((N_TASKS, D_LANG), lambda i: (0, 0)),
        ],
        out_specs=pl.BlockSpec((1, 1, rb), lambda i: (i, 0, 0)),
        out_shape=jax.ShapeDtypeStruct((nblk, 1, rb), jnp.int32),
    )(language_token, route_embeddings)
    return out.reshape(B)


def _pfx16(x):
    lanes = jax.lax.iota(jnp.int32, 16)
    for k in (1, 2, 4, 8):
        sh = x.at[jnp.maximum(lanes - k, 0)].get(mode="promise_in_bounds")
        x = x + jnp.where(lanes >= k, sh, 0)
    return x


def _bcast16(x, k):
    return x.at[jnp.full((16,), k, jnp.int32)].get(mode="promise_in_bounds")


def _sort_body(idx_hbm, inv_hbm, meta_hbm, idx_v, inv_v, meta_v):
    cid = jax.lax.axis_index("c")
    sid = jax.lax.axis_index("s")

    @pl.when(jnp.logical_and(cid == 0, sid == 0))
    def _():
        pltpu.sync_copy(idx_hbm, idx_v)
        lanes = jax.lax.iota(jnp.int32, 16)
        zeros = jnp.zeros((16,), jnp.int32)
        ones = jnp.ones((16,), jnp.int32)

        def h_body(c, accs):
            v = idx_v[pl.ds(c * 16, 16)]
            return tuple(acc + jnp.where(v == b, 1, 0)
                         for b, acc in enumerate(accs))

        accs = jax.lax.fori_loop(0, B // 16, h_body, (zeros,) * N_TASKS)
        counts = zeros
        for b in range(N_TASKS):
            tot = _bcast16(_pfx16(accs[b]), 15)
            counts = jnp.where(lanes == b, tot, counts)

        nblk = (counts + (TB - 1)) >> TB_LOG2
        bsi = _pfx16(nblk)
        pad_off = (bsi - nblk) * TB
        num_used_v = _bcast16(bsi, N_TASKS - 1)

        bsi_e = [_bcast16(bsi, e) for e in range(N_TASKS)]
        acc0 = zeros
        acc1 = zeros
        kvec1 = lanes + 16
        for e in range(N_TASKS):
            acc0 = acc0 + jnp.where(lanes >= bsi_e[e], 1, 0)
            acc1 = acc1 + jnp.where(kvec1 >= bsi_e[e], 1, 0)
        meta_v[pl.ds(0, 16)] = jnp.minimum(acc0, N_TASKS - 1)
        meta_v[pl.ds(16, 16)] = jnp.where(
            lanes == (NB_MAX - 16), num_used_v, jnp.minimum(acc1, N_TASKS - 1))

        def s_body(c, offs):
            v = idx_v[pl.ds(c * 16, 16)]
            lo = v < 4
            s0 = jnp.where(lo, v, 0) * 8
            s1 = jnp.where(lo, 0, v - 4) * 8
            w0 = jnp.where(lo, ones << s0, 0)
            w1 = jnp.where(lo, 0, ones << s1)
            p0 = _pfx16(w0)
            p1 = _pfx16(w1)
            psel = jnp.where(lo, p0, p1)
            rank = ((psel >> jnp.where(lo, s0, s1)) & 255) - 1
            off_v = offs.at[v].get(mode="promise_in_bounds")
            inv_v[pl.ds(c * 16, 16)] = off_v + rank
            t0 = _bcast16(p0, 15)
            t1 = _bcast16(p1, 15)
            c0 = (t0 >> jnp.minimum(lanes, 3) * 8) & 255
            c1 = (t1 >> (jnp.minimum(jnp.maximum(lanes - 4, 0), 3) * 8)) & 255
            cnt = jnp.where(lanes < 4, c0, jnp.where(lanes < 8, c1, 0))
            return offs + cnt

        jax.lax.fori_loop(0, B // 16, s_body, pad_off)

        pltpu.sync_copy(inv_v, inv_hbm)
        pltpu.sync_copy(meta_v, meta_hbm)


_sort = pl.kernel(
    _sort_body,
    out_type=[
        jax.ShapeDtypeStruct((B,), jnp.int32),
        jax.ShapeDtypeStruct((META_LEN,), jnp.int32),
    ],
    mesh=_SC_MESH,
    scratch_types=[
        pltpu.VMEM((B,), jnp.int32),
        pltpu.VMEM((B,), jnp.int32),
        pltpu.VMEM((META_LEN,), jnp.int32),
    ],
)


def _scatter_x_body(x_hbm, inv_hbm, xs_hbm, idx_c, rows_v, sem):
    cid = jax.lax.axis_index("c")
    sid = jax.lax.axis_index("s")
    base = (sid * 2 + cid) * (B // _NW)

    def c_body(j, carry):
        b2 = base + j * 32
        pltpu.sync_copy(inv_hbm.at[pl.ds(b2, 32)], idx_c)
        pltpu.sync_copy(x_hbm.at[pl.ds(b2, 32)], rows_v)
        pltpu.async_copy(rows_v, xs_hbm.at[idx_c], sem).wait()
        return carry

    jax.lax.fori_loop(0, (B // _NW) // 32, c_body, 0)


_scatter_x = pl.kernel(
    _scatter_x_body,
    out_type=jax.ShapeDtypeStruct((P, D_IN), jnp.float32),
    mesh=_SC_MESH,
    scratch_types=[
        pltpu.VMEM((32,), jnp.int32),
        pltpu.VMEM((32, D_IN), jnp.float32),
        pltpu.SemaphoreType.DMA,
    ],
)


def _gather_y_body(ys_hbm, inv_hbm, y_hbm, idx_c, rows_v, sem):
    cid = jax.lax.axis_index("c")
    sid = jax.lax.axis_index("s")
    base = (sid * 2 + cid) * (B // _NW)

    def c_body(j, carry):
        b2 = base + j * 32
        pltpu.sync_copy(inv_hbm.at[pl.ds(b2, 32)], idx_c)
        pltpu.async_copy(ys_hbm.at[idx_c], rows_v, sem).wait()
        pltpu.sync_copy(rows_v, y_hbm.at[pl.ds(b2, 32)])
        return carry

    jax.lax.fori_loop(0, (B // _NW) // 32, c_body, 0)


_gather_y = pl.kernel(
    _gather_y_body,
    out_type=jax.ShapeDtypeStruct((B, D_OUT), jnp.float32),
    mesh=_SC_MESH,
    scratch_types=[
        pltpu.VMEM((32,), jnp.int32),
        pltpu.VMEM((32, D_OUT), jnp.float32),
        pltpu.SemaphoreType.DMA,
    ],
)


def _mlp_body(meta_ref, x_ref, w1_ref, b1_ref, w2_ref, b2_ref, w3_ref, b3_ref,
              out_ref):
    i = pl.program_id(0)

    @pl.when(i < meta_ref[NB_MAX])
    def _():
        xb = x_ref[...].astype(jnp.bfloat16)
        h = jnp.maximum(
            jnp.dot(xb, w1_ref[0].astype(jnp.bfloat16),
                    preferred_element_type=jnp.float32)
            + b1_ref[0], 0.0)
        h = jnp.maximum(
            jnp.dot(h.astype(jnp.bfloat16), w2_ref[0].astype(jnp.bfloat16),
                    preferred_element_type=jnp.float32)
            + b2_ref[0], 0.0)
        out_ref[...] = jnp.tanh(
            jnp.dot(h.astype(jnp.bfloat16), w3_ref[0].astype(jnp.bfloat16),
                    preferred_element_type=jnp.float32)
            + b3_ref[0])


def _grouped_mlp(meta, x_sorted, W1, b1, W2, b2, W3, b3):
    grid_spec = pltpu.PrefetchScalarGridSpec(
        num_scalar_prefetch=1,
        grid=(NB_MAX,),
        in_specs=[
            pl.BlockSpec((TB, D_IN), lambda i, m: (i, 0)),
            pl.BlockSpec((1, D_IN, D_H), lambda i, m: (m[i], 0, 0)),
            pl.BlockSpec((1, 1, D_H), lambda i, m: (m[i], 0, 0)),
            pl.BlockSpec((1, D_H, D_H), lambda i, m: (m[i], 0, 0)),
            pl.BlockSpec((1, 1, D_H), lambda i, m: (m[i], 0, 0)),
            pl.BlockSpec((1, D_H, D_OUT), lambda i, m: (m[i], 0, 0)),
            pl.BlockSpec((1, 1, D_OUT), lambda i, m: (m[i], 0, 0)),
        ],
        out_specs=pl.BlockSpec((TB, D_OUT), lambda i, m: (i, 0)),
    )
    return pl.pallas_call(
        _mlp_body,
        grid_spec=grid_spec,
        out_shape=jax.ShapeDtypeStruct((P, D_OUT), jnp.float32),
        compiler_params=pltpu.CompilerParams(
            vmem_limit_bytes=100 * 1024 * 1024),
    )(meta, x_sorted,
      W1, b1.reshape(N_TASKS, 1, D_H),
      W2, b2.reshape(N_TASKS, 1, D_H),
      W3, b3.reshape(N_TASKS, 1, D_OUT))


def kernel(x, language_token, route_embeddings, W1, b1, W2, b2, W3, b3):
    indices = _route(language_token, route_embeddings)
    inv, meta = _sort(indices)
    x_sorted = _scatter_x(x, inv)
    y_sorted = _grouped_mlp(meta, x_sorted, W1, b1, W2, b2, W3, b3)
    return _gather_y(y_sorted, inv)


# speedup vs baseline: 1.2475x; 1.0227x over previous
import jax
import jax.numpy as jnp
from jax.experimental import pallas as pl
from jax.experimental.pallas import tpu as pltpu
from jax.experimental.pallas import tpu_sc as plsc

N_TASKS = 8
D_IN = 2048
D_OUT = 2048
D_H = 1024
D_LANG = 768
B = 4096

TB = 256
TB_LOG2 = 8
NB_MAX = B // TB + N_TASKS
P = NB_MAX * TB
META_LEN = 32

_SC_MESH = plsc.VectorSubcoreMesh(core_axis_name="c", subcore_axis_name="s")
_NW = 32


def _router_body(lt_ref, emb_ref, out_ref):
    lt = lt_ref[...]
    emb = emb_ref[...]
    nt = lt / jnp.maximum(
        jnp.sqrt(jnp.sum(lt * lt, axis=1, keepdims=True)), 1e-12)
    ne = emb / jnp.maximum(
        jnp.sqrt(jnp.sum(emb * emb, axis=1, keepdims=True)), 1e-12)
    sims = jax.lax.dot_general(
        nt, ne, (((1,), (1,)), ((), ())),
        preferred_element_type=jnp.float32)
    idx = jnp.argmax(sims, axis=1).astype(jnp.int32)
    out_ref[...] = idx[None, None, :]


def _route(language_token, route_embeddings):
    rb = 1024
    nblk = B // rb
    out = pl.pallas_call(
        _router_body,
        grid=(nblk,),
        in_specs=[
            pl.BlockSpec((rb, D_LANG), lambda i: (i, 0)),
            pl.BlockSpec((N_TASKS, D_LANG), lambda i: (0, 0)),
        ],
        out_specs=pl.BlockSpec((1, 1, rb), lambda i: (i, 0, 0)),
        out_shape=jax.ShapeDtypeStruct((nblk, 1, rb), jnp.int32),
    )(language_token, route_embeddings)
    return out.reshape(B)


def _pfx16(x):
    lanes = jax.lax.iota(jnp.int32, 16)
    for k in (1, 2, 4, 8):
        sh = x.at[jnp.maximum(lanes - k, 0)].get(mode="promise_in_bounds")
        x = x + jnp.where(lanes >= k, sh, 0)
    return x


def _bcast16(x, k):
    return x.at[jnp.full((16,), k, jnp.int32)].get(mode="promise_in_bounds")


def _sort_scatter_body(idx_hbm, x_hbm, xs_hbm, inv_hbm, meta_hbm,
                       idx_v, inv_v, meta_v, idx_c, rows_v, sem):
    cid = jax.lax.axis_index("c")
    sid = jax.lax.axis_index("s")
    wid = sid * 2 + cid

    pltpu.sync_copy(idx_hbm, idx_v)
    lanes = jax.lax.iota(jnp.int32, 16)
    zeros = jnp.zeros((16,), jnp.int32)
    ones = jnp.ones((16,), jnp.int32)

    if True:
        def h_body(c, accs):
            v = idx_v[pl.ds(c * 16, 16)]
            return tuple(acc + jnp.where(v == b, 1, 0)
                         for b, acc in enumerate(accs))

        accs = jax.lax.fori_loop(0, B // 16, h_body, (zeros,) * N_TASKS)
        counts = zeros
        for b in range(N_TASKS):
            tot = _bcast16(_pfx16(accs[b]), 15)
            counts = jnp.where(lanes == b, tot, counts)

        nblk = (counts + (TB - 1)) >> TB_LOG2
        bsi = _pfx16(nblk)
        pad_off = (bsi - nblk) * TB
        num_used_v = _bcast16(bsi, N_TASKS - 1)

        bsi_e = [_bcast16(bsi, e) for e in range(N_TASKS)]
        acc0 = zeros
        acc1 = zeros
        kvec1 = lanes + 16
        for e in range(N_TASKS):
            acc0 = acc0 + jnp.where(lanes >= bsi_e[e], 1, 0)
            acc1 = acc1 + jnp.where(kvec1 >= bsi_e[e], 1, 0)
        meta_v[pl.ds(0, 16)] = jnp.minimum(acc0, N_TASKS - 1)
        meta_v[pl.ds(16, 16)] = jnp.where(
            lanes == (NB_MAX - 16), num_used_v, jnp.minimum(acc1, N_TASKS - 1))

        def s_body(c, offs):
            v = idx_v[pl.ds(c * 16, 16)]
            lo = v < 4
            s0 = jnp.where(lo, v, 0) * 8
            s1 = jnp.where(lo, 0, v - 4) * 8
            w0 = jnp.where(lo, ones << s0, 0)
            w1 = jnp.where(lo, 0, ones << s1)
            p0 = _pfx16(w0)
            p1 = _pfx16(w1)
            psel = jnp.where(lo, p0, p1)
            rank = ((psel >> jnp.where(lo, s0, s1)) & 255) - 1
            off_v = offs.at[v].get(mode="promise_in_bounds")
            inv_v[pl.ds(c * 16, 16)] = off_v + rank
            t0 = _bcast16(p0, 15)
            t1 = _bcast16(p1, 15)
            c0 = (t0 >> jnp.minimum(lanes, 3) * 8) & 255
            c1 = (t1 >> (jnp.minimum(jnp.maximum(lanes - 4, 0), 3) * 8)) & 255
            cnt = jnp.where(lanes < 4, c0, jnp.where(lanes < 8, c1, 0))
            return offs + cnt

        jax.lax.fori_loop(0, B // 16, s_body, pad_off)

        @pl.when(wid == 0)
        def _():
            pltpu.sync_copy(meta_v, meta_hbm)

        base = wid * (B // _NW)
        pltpu.sync_copy(inv_v.at[pl.ds(base, B // _NW)],
                        inv_hbm.at[pl.ds(base, B // _NW)])

        def c_body(j, carry):
            b2 = base + j * 32
            pltpu.sync_copy(inv_hbm.at[pl.ds(b2, 32)], idx_c)
            pltpu.sync_copy(x_hbm.at[pl.ds(b2, 32)], rows_v)
            pltpu.async_copy(rows_v, xs_hbm.at[idx_c], sem).wait()
            return carry

        jax.lax.fori_loop(0, (B // _NW) // 32, c_body, 0)


_sort_scatter = pl.kernel(
    _sort_scatter_body,
    out_type=[
        jax.ShapeDtypeStruct((P, D_IN), jnp.float32),
        jax.ShapeDtypeStruct((B,), jnp.int32),
        jax.ShapeDtypeStruct((META_LEN,), jnp.int32),
    ],
    mesh=_SC_MESH,
    scratch_types=[
        pltpu.VMEM((B,), jnp.int32),
        pltpu.VMEM((B,), jnp.int32),
        pltpu.VMEM((META_LEN,), jnp.int32),
        pltpu.VMEM((32,), jnp.int32),
        pltpu.VMEM((32, D_IN), jnp.float32),
        pltpu.SemaphoreType.DMA,
    ],
)


def _gather_y_body(ys_hbm, inv_hbm, y_hbm, idx_c, rows_v, sem):
    cid = jax.lax.axis_index("c")
    sid = jax.lax.axis_index("s")
    base = (sid * 2 + cid) * (B // _NW)

    def c_body(j, carry):
        b2 = base + j * 32
        pltpu.sync_copy(inv_hbm.at[pl.ds(b2, 32)], idx_c)
        pltpu.async_copy(ys_hbm.at[idx_c], rows_v, sem).wait()
        pltpu.sync_copy(rows_v, y_hbm.at[pl.ds(b2, 32)])
        return carry

    jax.lax.fori_loop(0, (B // _NW) // 32, c_body, 0)


_gather_y = pl.kernel(
    _gather_y_body,
    out_type=jax.ShapeDtypeStruct((B, D_OUT), jnp.float32),
    mesh=_SC_MESH,
    scratch_types=[
        pltpu.VMEM((32,), jnp.int32),
        pltpu.VMEM((32, D_OUT), jnp.float32),
        pltpu.SemaphoreType.DMA,
    ],
)


def _mlp_body(meta_ref, x_ref, w1_ref, b1_ref, w2_ref, b2_ref, w3_ref, b3_ref,
              out_ref):
    i = pl.program_id(0)

    @pl.when(i < meta_ref[NB_MAX])
    def _():
        xb = x_ref[...].astype(jnp.bfloat16)
        h = jnp.maximum(
            jnp.dot(xb, w1_ref[0].astype(jnp.bfloat16),
                    preferred_element_type=jnp.float32)
            + b1_ref[0], 0.0)
        h = jnp.maximum(
            jnp.dot(h.astype(jnp.bfloat16), w2_ref[0].astype(jnp.bfloat16),
                    preferred_element_type=jnp.float32)
            + b2_ref[0], 0.0)
        out_ref[...] = jnp.tanh(
            jnp.dot(h.astype(jnp.bfloat16), w3_ref[0].astype(jnp.bfloat16),
                    preferred_element_type=jnp.float32)
            + b3_ref[0])


def _grouped_mlp(meta, x_sorted, W1, b1, W2, b2, W3, b3):
    grid_spec = pltpu.PrefetchScalarGridSpec(
        num_scalar_prefetch=1,
        grid=(NB_MAX,),
        in_specs=[
            pl.BlockSpec((TB, D_IN), lambda i, m: (i, 0)),
            pl.BlockSpec((1, D_IN, D_H), lambda i, m: (m[i], 0, 0)),
            pl.BlockSpec((1, 1, D_H), lambda i, m: (m[i], 0, 0)),
            pl.BlockSpec((1, D_H, D_H), lambda i, m: (m[i], 0, 0)),
            pl.BlockSpec((1, 1, D_H), lambda i, m: (m[i], 0, 0)),
            pl.BlockSpec((1, D_H, D_OUT), lambda i, m: (m[i], 0, 0)),
            pl.BlockSpec((1, 1, D_OUT), lambda i, m: (m[i], 0, 0)),
        ],
        out_specs=pl.BlockSpec((TB, D_OUT), lambda i, m: (i, 0)),
    )
    return pl.pallas_call(
        _mlp_body,
        grid_spec=grid_spec,
        out_shape=jax.ShapeDtypeStruct((P, D_OUT), jnp.float32),
        compiler_params=pltpu.CompilerParams(
            vmem_limit_bytes=100 * 1024 * 1024),
    )(meta, x_sorted,
      W1, b1.reshape(N_TASKS, 1, D_H),
      W2, b2.reshape(N_TASKS, 1, D_H),
      W3, b3.reshape(N_TASKS, 1, D_OUT))


def kernel(x, language_token, route_embeddings, W1, b1, W2, b2, W3, b3):
    indices = _route(language_token, route_embeddings)
    x_sorted, inv, meta = _sort_scatter(indices, x)
    y_sorted = _grouped_mlp(meta, x_sorted, W1, b1, W2, b2, W3, b3)
    return _gather_y(y_sorted, inv)


# speedup vs baseline: 1.2978x; 1.0403x over previous
import jax
import jax.numpy as jnp
from jax.experimental import pallas as pl
from jax.experimental.pallas import tpu as pltpu
from jax.experimental.pallas import tpu_sc as plsc

N_TASKS = 8
D_IN = 2048
D_OUT = 2048
D_H = 1024
D_LANG = 768
B = 4096

TB = 256
TB_LOG2 = 8
NB_MAX = B // TB + N_TASKS
P = NB_MAX * TB
META_LEN = 32

_SC_MESH = plsc.VectorSubcoreMesh(core_axis_name="c", subcore_axis_name="s")
_NW = 32


def _router_body(lt_ref, emb_ref, out_ref):
    lt = lt_ref[...]
    emb = emb_ref[...]
    nt = lt / jnp.maximum(
        jnp.sqrt(jnp.sum(lt * lt, axis=1, keepdims=True)), 1e-12)
    ne = emb / jnp.maximum(
        jnp.sqrt(jnp.sum(emb * emb, axis=1, keepdims=True)), 1e-12)
    sims = jax.lax.dot_general(
        nt, ne, (((1,), (1,)), ((), ())),
        preferred_element_type=jnp.float32)
    idx = jnp.argmax(sims, axis=1).astype(jnp.int32)
    out_ref[...] = idx[None, None, :]


def _route(language_token, route_embeddings):
    rb = 1024
    nblk = B // rb
    out = pl.pallas_call(
        _router_body,
        grid=(nblk,),
        in_specs=[
            pl.BlockSpec((rb, D_LANG), lambda i: (i, 0)),
            pl.BlockSpec((N_TASKS, D_LANG), lambda i: (0, 0)),
        ],
        out_specs=pl.BlockSpec((1, 1, rb), lambda i: (i, 0, 0)),
        out_shape=jax.ShapeDtypeStruct((nblk, 1, rb), jnp.int32),
    )(language_token, route_embeddings)
    return out.reshape(B)


def _pfx16(x):
    lanes = jax.lax.iota(jnp.int32, 16)
    for k in (1, 2, 4, 8):
        sh = x.at[jnp.maximum(lanes - k, 0)].get(mode="promise_in_bounds")
        x = x + jnp.where(lanes >= k, sh, 0)
    return x


def _bcast16(x, k):
    return x.at[jnp.full((16,), k, jnp.int32)].get(mode="promise_in_bounds")


def _sort_scatter_body(idx_hbm, x_hbm, xs_hbm, inv_hbm, meta_hbm,
                       idx_v, inv_v, meta_v, idx_c, rows_v, sem):
    cid = jax.lax.axis_index("c")
    sid = jax.lax.axis_index("s")
    wid = sid * 2 + cid

    pltpu.sync_copy(idx_hbm, idx_v)
    lanes = jax.lax.iota(jnp.int32, 16)
    zeros = jnp.zeros((16,), jnp.int32)
    ones = jnp.ones((16,), jnp.int32)

    if True:
        def h_body(c, accs):
            v = idx_v[pl.ds(c * 16, 16)]
            return tuple(acc + jnp.where(v == b, 1, 0)
                         for b, acc in enumerate(accs))

        accs = jax.lax.fori_loop(0, B // 16, h_body, (zeros,) * N_TASKS)
        counts = zeros
        for b in range(N_TASKS):
            tot = _bcast16(_pfx16(accs[b]), 15)
            counts = jnp.where(lanes == b, tot, counts)

        nblk = (counts + (TB - 1)) >> TB_LOG2
        bsi = _pfx16(nblk)
        pad_off = (bsi - nblk) * TB
        num_used_v = _bcast16(bsi, N_TASKS - 1)

        bsi_e = [_bcast16(bsi, e) for e in range(N_TASKS)]
        acc0 = zeros
        acc1 = zeros
        kvec0 = jnp.minimum(lanes, num_used_v - 1)
        kvec1 = jnp.minimum(lanes + 16, num_used_v - 1)
        for e in range(N_TASKS):
            acc0 = acc0 + jnp.where(kvec0 >= bsi_e[e], 1, 0)
            acc1 = acc1 + jnp.where(kvec1 >= bsi_e[e], 1, 0)
        meta_v[pl.ds(0, 16)] = jnp.minimum(acc0, N_TASKS - 1)
        meta_v[pl.ds(16, 16)] = jnp.where(
            lanes == (NB_MAX - 16), num_used_v, jnp.minimum(acc1, N_TASKS - 1))

        def s_body(c, offs):
            v = idx_v[pl.ds(c * 16, 16)]
            lo = v < 4
            s0 = jnp.where(lo, v, 0) * 8
            s1 = jnp.where(lo, 0, v - 4) * 8
            w0 = jnp.where(lo, ones << s0, 0)
            w1 = jnp.where(lo, 0, ones << s1)
            p0 = _pfx16(w0)
            p1 = _pfx16(w1)
            psel = jnp.where(lo, p0, p1)
            rank = ((psel >> jnp.where(lo, s0, s1)) & 255) - 1
            off_v = offs.at[v].get(mode="promise_in_bounds")
            inv_v[pl.ds(c * 16, 16)] = off_v + rank
            t0 = _bcast16(p0, 15)
            t1 = _bcast16(p1, 15)
            c0 = (t0 >> jnp.minimum(lanes, 3) * 8) & 255
            c1 = (t1 >> (jnp.minimum(jnp.maximum(lanes - 4, 0), 3) * 8)) & 255
            cnt = jnp.where(lanes < 4, c0, jnp.where(lanes < 8, c1, 0))
            return offs + cnt

        jax.lax.fori_loop(0, B // 16, s_body, pad_off)

        @pl.when(wid == 0)
        def _():
            pltpu.sync_copy(meta_v, meta_hbm)

        base = wid * (B // _NW)
        pltpu.sync_copy(inv_v.at[pl.ds(base, B // _NW)],
                        inv_hbm.at[pl.ds(base, B // _NW)])

        def c_body(j, carry):
            b2 = base + j * 32
            pltpu.sync_copy(inv_hbm.at[pl.ds(b2, 32)], idx_c)
            pltpu.sync_copy(x_hbm.at[pl.ds(b2, 32)], rows_v)
            pltpu.async_copy(rows_v, xs_hbm.at[idx_c], sem).wait()
            return carry

        jax.lax.fori_loop(0, (B // _NW) // 32, c_body, 0)


_sort_scatter = pl.kernel(
    _sort_scatter_body,
    out_type=[
        jax.ShapeDtypeStruct((P, D_IN), jnp.float32),
        jax.ShapeDtypeStruct((B,), jnp.int32),
        jax.ShapeDtypeStruct((META_LEN,), jnp.int32),
    ],
    mesh=_SC_MESH,
    scratch_types=[
        pltpu.VMEM((B,), jnp.int32),
        pltpu.VMEM((B,), jnp.int32),
        pltpu.VMEM((META_LEN,), jnp.int32),
        pltpu.VMEM((32,), jnp.int32),
        pltpu.VMEM((32, D_IN), jnp.float32),
        pltpu.SemaphoreType.DMA,
    ],
)


def _gather_y_body(ys_hbm, inv_hbm, y_hbm, idx_c, rows_v, sem):
    cid = jax.lax.axis_index("c")
    sid = jax.lax.axis_index("s")
    base = (sid * 2 + cid) * (B // _NW)

    def c_body(j, carry):
        b2 = base + j * 32
        pltpu.sync_copy(inv_hbm.at[pl.ds(b2, 32)], idx_c)
        pltpu.async_copy(ys_hbm.at[idx_c], rows_v, sem).wait()
        pltpu.sync_copy(rows_v, y_hbm.at[pl.ds(b2, 32)])
        return carry

    jax.lax.fori_loop(0, (B // _NW) // 32, c_body, 0)


_gather_y = pl.kernel(
    _gather_y_body,
    out_type=jax.ShapeDtypeStruct((B, D_OUT), jnp.float32),
    mesh=_SC_MESH,
    scratch_types=[
        pltpu.VMEM((32,), jnp.int32),
        pltpu.VMEM((32, D_OUT), jnp.float32),
        pltpu.SemaphoreType.DMA,
    ],
)


def _mlp_body(meta_ref, x_ref, w1_ref, b1_ref, w2_ref, b2_ref, w3_ref, b3_ref,
              out_ref):
    i = pl.program_id(0)

    @pl.when(i < meta_ref[NB_MAX])
    def _():
        xb = x_ref[...].astype(jnp.bfloat16)
        h = jnp.maximum(
            jnp.dot(xb, w1_ref[0].astype(jnp.bfloat16),
                    preferred_element_type=jnp.float32)
            + b1_ref[0], 0.0)
        h = jnp.maximum(
            jnp.dot(h.astype(jnp.bfloat16), w2_ref[0].astype(jnp.bfloat16),
                    preferred_element_type=jnp.float32)
            + b2_ref[0], 0.0)
        out_ref[...] = jnp.tanh(
            jnp.dot(h.astype(jnp.bfloat16), w3_ref[0].astype(jnp.bfloat16),
                    preferred_element_type=jnp.float32)
            + b3_ref[0])


def _grouped_mlp(meta, x_sorted, W1, b1, W2, b2, W3, b3):
    grid_spec = pltpu.PrefetchScalarGridSpec(
        num_scalar_prefetch=1,
        grid=(NB_MAX,),
        in_specs=[
            pl.BlockSpec((TB, D_IN),
                         lambda i, m: (jnp.minimum(i, m[NB_MAX] - 1), 0)),
            pl.BlockSpec((1, D_IN, D_H), lambda i, m: (m[i], 0, 0)),
            pl.BlockSpec((1, 1, D_H), lambda i, m: (m[i], 0, 0)),
            pl.BlockSpec((1, D_H, D_H), lambda i, m: (m[i], 0, 0)),
            pl.BlockSpec((1, 1, D_H), lambda i, m: (m[i], 0, 0)),
            pl.BlockSpec((1, D_H, D_OUT), lambda i, m: (m[i], 0, 0)),
            pl.BlockSpec((1, 1, D_OUT), lambda i, m: (m[i], 0, 0)),
        ],
        out_specs=pl.BlockSpec(
            (TB, D_OUT), lambda i, m: (jnp.minimum(i, m[NB_MAX] - 1), 0)),
    )
    return pl.pallas_call(
        _mlp_body,
        grid_spec=grid_spec,
        out_shape=jax.ShapeDtypeStruct((P, D_OUT), jnp.float32),
        compiler_params=pltpu.CompilerParams(
            vmem_limit_bytes=100 * 1024 * 1024),
    )(meta, x_sorted,
      W1, b1.reshape(N_TASKS, 1, D_H),
      W2, b2.reshape(N_TASKS, 1, D_H),
      W3, b3.reshape(N_TASKS, 1, D_OUT))


def kernel(x, language_token, route_embeddings, W1, b1, W2, b2, W3, b3):
    indices = _route(language_token, route_embeddings)
    x_sorted, inv, meta = _sort_scatter(indices, x)
    y_sorted = _grouped_mlp(meta, x_sorted, W1, b1, W2, b2, W3, b3)
    return _gather_y(y_sorted, inv)


# speedup vs baseline: 1.3004x; 1.0020x over previous
import jax
import jax.numpy as jnp
from jax.experimental import pallas as pl
from jax.experimental.pallas import tpu as pltpu
from jax.experimental.pallas import tpu_sc as plsc

N_TASKS = 8
D_IN = 2048
D_OUT = 2048
D_H = 1024
D_LANG = 768
B = 4096

TB = 256
TB_LOG2 = 8
NB_MAX = B // TB + N_TASKS
P = NB_MAX * TB
META_LEN = 32

_SC_MESH = plsc.VectorSubcoreMesh(core_axis_name="c", subcore_axis_name="s")
_NW = 32


def _router_body(lt_ref, emb_ref, out_ref):
    lt = lt_ref[...]
    emb = emb_ref[...]
    nt = lt / jnp.maximum(
        jnp.sqrt(jnp.sum(lt * lt, axis=1, keepdims=True)), 1e-12)
    ne = emb / jnp.maximum(
        jnp.sqrt(jnp.sum(emb * emb, axis=1, keepdims=True)), 1e-12)
    sims = jax.lax.dot_general(
        nt, ne, (((1,), (1,)), ((), ())),
        preferred_element_type=jnp.float32)
    idx = jnp.argmax(sims, axis=1).astype(jnp.int32)
    out_ref[...] = idx[None, None, :]


def _route(language_token, route_embeddings):
    rb = 1024
    nblk = B // rb
    out = pl.pallas_call(
        _router_body,
        grid=(nblk,),
        in_specs=[
            pl.BlockSpec((rb, D_LANG), lambda i: (i, 0)),
            pl.BlockSpec((N_TASKS, D_LANG), lambda i: (0, 0)),
        ],
        out_specs=pl.BlockSpec((1, 1, rb), lambda i: (i, 0, 0)),
        out_shape=jax.ShapeDtypeStruct((nblk, 1, rb), jnp.int32),
    )(language_token, route_embeddings)
    return out.reshape(B)


def _pfx16(x):
    lanes = jax.lax.iota(jnp.int32, 16)
    for k in (1, 2, 4, 8):
        sh = x.at[jnp.maximum(lanes - k, 0)].get(mode="promise_in_bounds")
        x = x + jnp.where(lanes >= k, sh, 0)
    return x


def _bcast16(x, k):
    return x.at[jnp.full((16,), k, jnp.int32)].get(mode="promise_in_bounds")


def _sort_scatter_body(idx_hbm, x_hbm, xs_hbm, inv_hbm, meta_hbm,
                       idx_v, inv_v, meta_v, idx_c, rows_v, sem):
    cid = jax.lax.axis_index("c")
    sid = jax.lax.axis_index("s")
    wid = sid * 2 + cid

    pltpu.sync_copy(idx_hbm, idx_v)
    lanes = jax.lax.iota(jnp.int32, 16)
    zeros = jnp.zeros((16,), jnp.int32)
    ones = jnp.ones((16,), jnp.int32)

    if True:
        def h_body(c, accs):
            v = idx_v[pl.ds(c * 16, 16)]
            return tuple(acc + jnp.where(v == b, 1, 0)
                         for b, acc in enumerate(accs))

        accs = jax.lax.fori_loop(0, B // 16, h_body, (zeros,) * N_TASKS)
        counts = zeros
        for b in range(N_TASKS):
            tot = _bcast16(_pfx16(accs[b]), 15)
            counts = jnp.where(lanes == b, tot, counts)

        nblk = (counts + (TB - 1)) >> TB_LOG2
        bsi = _pfx16(nblk)
        pad_off = (bsi - nblk) * TB
        num_used_v = _bcast16(bsi, N_TASKS - 1)

        bsi_e = [_bcast16(bsi, e) for e in range(N_TASKS)]
        acc0 = zeros
        acc1 = zeros
        kvec0 = jnp.minimum(lanes, num_used_v - 1)
        kvec1 = jnp.minimum(lanes + 16, num_used_v - 1)
        for e in range(N_TASKS):
            acc0 = acc0 + jnp.where(kvec0 >= bsi_e[e], 1, 0)
            acc1 = acc1 + jnp.where(kvec1 >= bsi_e[e], 1, 0)
        meta_v[pl.ds(0, 16)] = jnp.minimum(acc0, N_TASKS - 1)
        meta_v[pl.ds(16, 16)] = jnp.where(
            lanes == (NB_MAX - 16), num_used_v, jnp.minimum(acc1, N_TASKS - 1))

        def s_body(c, offs):
            v = idx_v[pl.ds(c * 16, 16)]
            lo = v < 4
            s0 = jnp.where(lo, v, 0) * 8
            s1 = jnp.where(lo, 0, v - 4) * 8
            w0 = jnp.where(lo, ones << s0, 0)
            w1 = jnp.where(lo, 0, ones << s1)
            p0 = _pfx16(w0)
            p1 = _pfx16(w1)
            psel = jnp.where(lo, p0, p1)
            rank = ((psel >> jnp.where(lo, s0, s1)) & 255) - 1
            off_v = offs.at[v].get(mode="promise_in_bounds")
            inv_v[pl.ds(c * 16, 16)] = off_v + rank
            t0 = _bcast16(p0, 15)
            t1 = _bcast16(p1, 15)
            c0 = (t0 >> jnp.minimum(lanes, 3) * 8) & 255
            c1 = (t1 >> (jnp.minimum(jnp.maximum(lanes - 4, 0), 3) * 8)) & 255
            cnt = jnp.where(lanes < 4, c0, jnp.where(lanes < 8, c1, 0))
            return offs + cnt

        jax.lax.fori_loop(0, B // 16, s_body, pad_off)

        @pl.when(wid == 0)
        def _():
            pltpu.sync_copy(meta_v, meta_hbm)

        base = wid * (B // _NW)
        pltpu.sync_copy(inv_v.at[pl.ds(base, B // _NW)],
                        inv_hbm.at[pl.ds(base, B // _NW)])

        def c_body(j, carry):
            b2 = base + j * 32
            pltpu.sync_copy(inv_hbm.at[pl.ds(b2, 32)], idx_c)
            pltpu.sync_copy(x_hbm.at[pl.ds(b2, 32)], rows_v)
            pltpu.async_copy(rows_v, xs_hbm.at[idx_c], sem).wait()
            return carry

        jax.lax.fori_loop(0, (B // _NW) // 32, c_body, 0)


_sort_scatter = pl.kernel(
    _sort_scatter_body,
    out_type=[
        jax.ShapeDtypeStruct((P, D_IN), jnp.float32),
        jax.ShapeDtypeStruct((B,), jnp.int32),
        jax.ShapeDtypeStruct((META_LEN,), jnp.int32),
    ],
    mesh=_SC_MESH,
    scratch_types=[
        pltpu.VMEM((B,), jnp.int32),
        pltpu.VMEM((B,), jnp.int32),
        pltpu.VMEM((META_LEN,), jnp.int32),
        pltpu.VMEM((32,), jnp.int32),
        pltpu.VMEM((32, D_IN), jnp.float32),
        pltpu.SemaphoreType.DMA,
    ],
)


def _gather_y_body(ys_hbm, inv_hbm, y_hbm, idx_c, rows_v, sem):
    cid = jax.lax.axis_index("c")
    sid = jax.lax.axis_index("s")
    base = (sid * 2 + cid) * (B // _NW)

    def c_body(j, carry):
        b2 = base + j * 32
        pltpu.sync_copy(inv_hbm.at[pl.ds(b2, 32)], idx_c)
        pltpu.async_copy(ys_hbm.at[idx_c], rows_v, sem).wait()
        pltpu.sync_copy(rows_v, y_hbm.at[pl.ds(b2, 32)])
        return carry

    jax.lax.fori_loop(0, (B // _NW) // 32, c_body, 0)


_gather_y = pl.kernel(
    _gather_y_body,
    out_type=jax.ShapeDtypeStruct((B, D_OUT), jnp.float32),
    mesh=_SC_MESH,
    scratch_types=[
        pltpu.VMEM((32,), jnp.int32),
        pltpu.VMEM((32, D_OUT), jnp.float32),
        pltpu.SemaphoreType.DMA,
    ],
)


def _mlp_body(meta_ref, x_ref, w1_ref, b1_ref, w2_ref, b2_ref, w3_ref, b3_ref,
              out_ref):
    i = pl.program_id(0)

    @pl.when(i < meta_ref[NB_MAX])
    def _():
        xb = x_ref[...]
        h = jnp.maximum(
            jnp.dot(xb, w1_ref[0], preferred_element_type=jnp.float32)
            + b1_ref[0], 0.0)
        h = jnp.maximum(
            jnp.dot(h, w2_ref[0], preferred_element_type=jnp.float32)
            + b2_ref[0], 0.0)
        out_ref[...] = jnp.tanh(
            jnp.dot(h, w3_ref[0], preferred_element_type=jnp.float32)
            + b3_ref[0])


def _grouped_mlp(meta, x_sorted, W1, b1, W2, b2, W3, b3):
    grid_spec = pltpu.PrefetchScalarGridSpec(
        num_scalar_prefetch=1,
        grid=(NB_MAX,),
        in_specs=[
            pl.BlockSpec((TB, D_IN),
                         lambda i, m: (jnp.minimum(i, m[NB_MAX] - 1), 0)),
            pl.BlockSpec((1, D_IN, D_H), lambda i, m: (m[i], 0, 0)),
            pl.BlockSpec((1, 1, D_H), lambda i, m: (m[i], 0, 0)),
            pl.BlockSpec((1, D_H, D_H), lambda i, m: (m[i], 0, 0)),
            pl.BlockSpec((1, 1, D_H), lambda i, m: (m[i], 0, 0)),
            pl.BlockSpec((1, D_H, D_OUT), lambda i, m: (m[i], 0, 0)),
            pl.BlockSpec((1, 1, D_OUT), lambda i, m: (m[i], 0, 0)),
        ],
        out_specs=pl.BlockSpec(
            (TB, D_OUT), lambda i, m: (jnp.minimum(i, m[NB_MAX] - 1), 0)),
    )
    return pl.pallas_call(
        _mlp_body,
        grid_spec=grid_spec,
        out_shape=jax.ShapeDtypeStruct((P, D_OUT), jnp.float32),
        compiler_params=pltpu.CompilerParams(
            vmem_limit_bytes=100 * 1024 * 1024),
    )(meta, x_sorted,
      W1, b1.reshape(N_TASKS, 1, D_H),
      W2, b2.reshape(N_TASKS, 1, D_H),
      W3, b3.reshape(N_TASKS, 1, D_OUT))


def kernel(x, language_token, route_embeddings, W1, b1, W2, b2, W3, b3):
    indices = _route(language_token, route_embeddings)
    x_sorted, inv, meta = _sort_scatter(indices, x)
    y_sorted = _grouped_mlp(meta, x_sorted, W1, b1, W2, b2, W3, b3)
    return _gather_y(y_sorted, inv)
